# Initial kernel scaffold; baseline (speedup 1.0000x reference)
import jax
import jax.numpy as jnp
from jax.experimental import pallas as pl


def kernel(x, edge_index, edge_weights, W_rel0, W_root0, b0, W_rel1, W_root1, b1, W_rel2, W_root2, b2):
    raise NotImplementedError("write your pallas kernel here")


# speedup vs baseline: 7.1046x; 7.1046x over previous
import functools

import jax
import jax.numpy as jnp
from jax import lax
from jax.experimental import pallas as pl
from jax.experimental.pallas import tpu as pltpu
from jax.experimental.pallas import tpu_sc as plsc

N = 10000
E = 320000
D_H = 32
D_OUT = 64

NC = 2
NS = 16
NW = NC * NS
CG = 8
COLS = D_H // CG
EG = NW // CG
EPW = E // EG
CH = 8000
NCHUNK = EPW // CH

_mesh = plsc.VectorSubcoreMesh(
    core_axis_name="c", subcore_axis_name="s", num_cores=NC, num_subcores=NS)


@functools.partial(
    pl.kernel,
    out_type=jax.ShapeDtypeStruct((EG * D_H * N,), jnp.float32),
    mesh=_mesh,
    compiler_params=pltpu.CompilerParams(
        use_tc_tiling_on_sc=False, needs_layout_passes=False),
    scratch_types=[
        pltpu.VMEM((COLS * N,), jnp.float32),
        pltpu.VMEM((COLS * N,), jnp.float32),
        pltpu.VMEM((CH,), jnp.int32),
        pltpu.VMEM((CH,), jnp.int32),
        pltpu.VMEM((CH,), jnp.float32),
    ],
)
def _sc_scatter(pt_hbm, src_hbm, dst_hbm, ew_hbm, out_hbm,
                p_sl, a_sl, s_buf, d_buf, w_buf):
    wid = lax.axis_index("s") * NC + lax.axis_index("c")
    cc = wid % CG
    g = wid // CG
    col0 = cc * COLS
    e0 = g * EPW

    pltpu.sync_copy(pt_hbm.at[pl.ds(col0 * N, COLS * N)], p_sl)

    def _zero(i, _):
        a_sl[pl.ds(i * 16, 16)] = jnp.zeros((16,), jnp.float32)
        return _
    lax.fori_loop(0, COLS * N // 16, _zero, None)

    def _chunk(k, _):
        base = e0 + k * CH
        pltpu.sync_copy(src_hbm.at[pl.ds(base, CH)], s_buf)
        pltpu.sync_copy(dst_hbm.at[pl.ds(base, CH)], d_buf)
        pltpu.sync_copy(ew_hbm.at[pl.ds(base, CH)], w_buf)

        def _edges(t, _):
            sl = pl.ds(t * 16, 16)
            si = s_buf[sl]
            di = d_buf[sl]
            wv = w_buf[sl]
            for j in range(COLS):
                vals = plsc.load_gather(p_sl, [si + (j * N)])
                plsc.addupdate_scatter(a_sl, [di + (j * N)], vals * wv)
            return _
        lax.fori_loop(0, CH // 16, _edges, None)
        return _
    lax.fori_loop(0, NCHUNK, _chunk, None)

    pltpu.sync_copy(a_sl, out_hbm.at[pl.ds(g * (D_H * N) + col0 * N, COLS * N)])


_DN0 = (((0,), (1,)), ((), ()))
_DNT = (((0,), (0,)), ((), ()))


def _tc1_body(x_ref, wr_ref, wo_ref, pt_ref, rt_ref):
    x = x_ref[...]
    pt_ref[...] = lax.dot_general(wr_ref[...], x, _DN0,
                                  preferred_element_type=jnp.float32)
    rt_ref[...] = lax.dot_general(wo_ref[...], x, _DN0,
                                  preferred_element_type=jnp.float32)


def _leaky(h):
    return jnp.where(h >= 0.0, h, h * 0.01)


def _tc2_body(ap_ref, rt_ref, b_ref, wr_ref, wo_ref, pt_ref, rt2_ref):
    h = _leaky(jnp.sum(ap_ref[...], axis=0) + rt_ref[...] + b_ref[...])
    pt_ref[...] = lax.dot_general(wr_ref[...], h, _DNT,
                                  preferred_element_type=jnp.float32)
    rt2_ref[...] = lax.dot_general(wo_ref[...], h, _DNT,
                                   preferred_element_type=jnp.float32)


def _tc3_body(ap_ref, rt_ref, b_ref, h_ref):
    h_ref[...] = _leaky(jnp.sum(ap_ref[...], axis=0) + rt_ref[...] + b_ref[...])


def _tc4_body(ap_ref, h_ref, wr_ref, wo_ref, b_ref, out_ref):
    agg = jnp.sum(ap_ref[...], axis=0)
    out_ref[...] = (
        lax.dot_general(agg, wr_ref[...], _DNT,
                        preferred_element_type=jnp.float32)
        + lax.dot_general(h_ref[...], wo_ref[...], _DNT,
                          preferred_element_type=jnp.float32)
        + b_ref[...])


def _f32(shape):
    return jax.ShapeDtypeStruct(shape, jnp.float32)


def kernel(x, edge_index, edge_weights,
           W_rel0, W_root0, b0,
           W_rel1, W_root1, b1,
           W_rel2, W_root2, b2):
    src = edge_index[0].astype(jnp.int32)
    dst = edge_index[1].astype(jnp.int32)
    ew = edge_weights.astype(jnp.float32)

    p0t, r0t = pl.pallas_call(
        _tc1_body,
        out_shape=(_f32((D_H, N)), _f32((D_H, N))),
    )(x, W_rel0, W_root0)

    a0p = _sc_scatter(p0t.reshape(-1), src, dst, ew).reshape(EG, D_H, N)

    p1t, r1t = pl.pallas_call(
        _tc2_body,
        out_shape=(_f32((D_H, N)), _f32((D_H, N))),
    )(a0p, r0t, b0.reshape(D_H, 1), W_rel1, W_root1)

    a1p = _sc_scatter(p1t.reshape(-1), src, dst, ew).reshape(EG, D_H, N)

    h2t = pl.pallas_call(
        _tc3_body,
        out_shape=_f32((D_H, N)),
    )(a1p, r1t, b1.reshape(D_H, 1))

    a2p = _sc_scatter(h2t.reshape(-1), src, dst, ew).reshape(EG, D_H, N)

    out = pl.pallas_call(
        _tc4_body,
        out_shape=_f32((N, D_OUT)),
    )(a2p, h2t, W_rel2, W_root2, b2.reshape(1, D_OUT))
    return out


# speedup vs baseline: 17.2535x; 2.4285x over previous
import functools

import jax
import jax.numpy as jnp
from jax import lax
from jax.experimental import pallas as pl
from jax.experimental.pallas import tpu as pltpu
from jax.experimental.pallas import tpu_sc as plsc

N = 10000
E = 320000
D_H = 32
D_OUT = 64

NC = 2
NS = 16
NW = NC * NS
CG = 8
COLS = D_H // CG
EG = NW // CG
EPW = E // EG
CH = 8000
NCHUNK = EPW // CH
UNROLL = 5

_mesh = plsc.VectorSubcoreMesh(
    core_axis_name="c", subcore_axis_name="s", num_cores=NC, num_subcores=NS)


@functools.partial(
    pl.kernel,
    out_type=jax.ShapeDtypeStruct((EG * D_H * N,), jnp.float32),
    mesh=_mesh,
    compiler_params=pltpu.CompilerParams(
        use_tc_tiling_on_sc=False, needs_layout_passes=False),
    scratch_types=[
        pltpu.VMEM((COLS * N,), jnp.float32),
        pltpu.VMEM((COLS * N,), jnp.float32),
        pltpu.VMEM((2, CH), jnp.int32),
        pltpu.VMEM((2, CH), jnp.int32),
        pltpu.VMEM((2, CH), jnp.float32),
        pltpu.SemaphoreType.DMA,
        pltpu.SemaphoreType.DMA,
    ],
)
def _sc_scatter(pt_hbm, src_hbm, dst_hbm, ew_hbm, out_hbm,
                p_sl, a_sl, s_buf, d_buf, w_buf, sem0, sem1):
    wid = lax.axis_index("s") * NC + lax.axis_index("c")
    cc = wid % CG
    g = wid // CG
    col0 = cc * COLS
    e0 = g * EPW

    pltpu.sync_copy(pt_hbm.at[pl.ds(col0 * N, COLS * N)], p_sl)

    @plsc.parallel_loop(0, COLS * N // 16, 1, unroll=10)
    def _zero(i):
        a_sl[pl.ds(i * 16, 16)] = jnp.zeros((16,), jnp.float32)

    sems = (sem0, sem1)

    def _start(k, slot):
        base = e0 + k * CH
        return [
            pltpu.async_copy(src_hbm.at[pl.ds(base, CH)], s_buf.at[slot],
                             sems[slot]),
            pltpu.async_copy(dst_hbm.at[pl.ds(base, CH)], d_buf.at[slot],
                             sems[slot]),
            pltpu.async_copy(ew_hbm.at[pl.ds(base, CH)], w_buf.at[slot],
                             sems[slot]),
        ]

    pend = {0: _start(0, 0)}
    for k in range(NCHUNK):
        if k + 1 < NCHUNK:
            pend[k + 1] = _start(k + 1, (k + 1) % 2)
        for dsc in pend.pop(k):
            dsc.wait()
        slot = k % 2

        @plsc.parallel_loop(0, CH // 16, 1, unroll=UNROLL)
        def _edges(t, slot=slot):
            sl = pl.ds(t * 16, 16)
            si = s_buf[slot, sl]
            di = d_buf[slot, sl]
            wv = w_buf[slot, sl]
            for j in range(COLS):
                vals = plsc.load_gather(p_sl, [si + (j * N)])
                plsc.addupdate_scatter(a_sl, [di + (j * N)], vals * wv)

    pltpu.sync_copy(a_sl, out_hbm.at[pl.ds(g * (D_H * N) + col0 * N, COLS * N)])


_DN0 = (((0,), (1,)), ((), ()))
_DNT = (((0,), (0,)), ((), ()))


def _tc1_body(x_ref, wr_ref, wo_ref, pt_ref, rt_ref):
    x = x_ref[...]
    pt_ref[...] = lax.dot_general(wr_ref[...], x, _DN0,
                                  preferred_element_type=jnp.float32)
    rt_ref[...] = lax.dot_general(wo_ref[...], x, _DN0,
                                  preferred_element_type=jnp.float32)


def _leaky(h):
    return jnp.where(h >= 0.0, h, h * 0.01)


def _tc2_body(ap_ref, rt_ref, b_ref, wr_ref, wo_ref, pt_ref, rt2_ref):
    h = _leaky(jnp.sum(ap_ref[...], axis=0) + rt_ref[...] + b_ref[...])
    pt_ref[...] = lax.dot_general(wr_ref[...], h, _DNT,
                                  preferred_element_type=jnp.float32)
    rt2_ref[...] = lax.dot_general(wo_ref[...], h, _DNT,
                                   preferred_element_type=jnp.float32)


def _tc3_body(ap_ref, rt_ref, b_ref, h_ref):
    h_ref[...] = _leaky(jnp.sum(ap_ref[...], axis=0) + rt_ref[...] + b_ref[...])


def _tc4_body(ap_ref, h_ref, wr_ref, wo_ref, b_ref, out_ref):
    agg = jnp.sum(ap_ref[...], axis=0)
    out_ref[...] = (
        lax.dot_general(agg, wr_ref[...], _DNT,
                        preferred_element_type=jnp.float32)
        + lax.dot_general(h_ref[...], wo_ref[...], _DNT,
                          preferred_element_type=jnp.float32)
        + b_ref[...])


def _f32(shape):
    return jax.ShapeDtypeStruct(shape, jnp.float32)


def kernel(x, edge_index, edge_weights,
           W_rel0, W_root0, b0,
           W_rel1, W_root1, b1,
           W_rel2, W_root2, b2):
    src = edge_index[0].astype(jnp.int32)
    dst = edge_index[1].astype(jnp.int32)
    ew = edge_weights.astype(jnp.float32)

    p0t, r0t = pl.pallas_call(
        _tc1_body,
        out_shape=(_f32((D_H, N)), _f32((D_H, N))),
    )(x, W_rel0, W_root0)

    a0p = _sc_scatter(p0t.reshape(-1), src, dst, ew).reshape(EG, D_H, N)

    p1t, r1t = pl.pallas_call(
        _tc2_body,
        out_shape=(_f32((D_H, N)), _f32((D_H, N))),
    )(a0p, r0t, b0.reshape(D_H, 1), W_rel1, W_root1)

    a1p = _sc_scatter(p1t.reshape(-1), src, dst, ew).reshape(EG, D_H, N)

    h2t = pl.pallas_call(
        _tc3_body,
        out_shape=_f32((D_H, N)),
    )(a1p, r1t, b1.reshape(D_H, 1))

    a2p = _sc_scatter(h2t.reshape(-1), src, dst, ew).reshape(EG, D_H, N)

    out = pl.pallas_call(
        _tc4_body,
        out_shape=_f32((N, D_OUT)),
    )(a2p, h2t, W_rel2, W_root2, b2.reshape(1, D_OUT))
    return out
